# baseline (speedup 1.0000x reference)
import functools

import jax
import jax.numpy as jnp
from jax import lax
from jax.experimental import pallas as pl
from jax.experimental.pallas import tpu as pltpu
from jax.experimental.pallas import tpu_sc as plsc

N = 8192
K = 20
ROWS = 256
CHUNK = 512
NCHUNK = N // CHUNK
NBLK = N // ROWS
BIG = 2 ** 30

SC_NC = 2
SC_NS = 16
SC_NW = SC_NC * SC_NS
PER_W = N // SC_NW
GCH = 128


def _topk_body(featp_ref, rowmeta_ref, wsub_ref, wb_ref, b_ref,
               idx_ref, c_ref, p_ref):
    g = pl.program_id(0)
    feat = featp_ref[pl.ds(g * ROWS, ROWS), :]
    c_ref[...] = jnp.dot(feat, wsub_ref[...],
                         preferred_element_type=jnp.float32) + b_ref[...]
    p_ref[...] = jnp.dot(feat, wb_ref[...],
                         preferred_element_type=jnp.float32)

    lo = rowmeta_ref[:, 0:1]
    hi = rowmeta_ref[:, 1:2]
    row_ids = g * ROWS + lax.broadcasted_iota(jnp.int32, (ROWS, 1), 0)

    cb_lo = jnp.min(lo) // CHUNK
    cb_hi = (jnp.max(hi) + CHUNK - 1) // CHUNK
    ones = jnp.full((1, 128), 1.0, jnp.float32)

    def chunk_step(t, carry):
        vals, idxs = carry
        cb = cb_lo + t
        fchunk = featp_ref[pl.ds(cb * CHUNK, CHUNK), :]
        dot = lax.dot_general(feat, fchunk, (((1,), (1,)), ((), ())),
                              preferred_element_type=jnp.float32)
        sqj = lax.dot_general(ones, fchunk * fchunk, (((1,), (1,)), ((), ())),
                              preferred_element_type=jnp.float32)
        neg = 2.0 * dot - sqj
        col_ids = cb * CHUNK + lax.broadcasted_iota(jnp.int32, (1, CHUNK), 1)
        valid = (col_ids >= lo) & (col_ids < hi) & (col_ids != row_ids)
        negm = jnp.where(valid, neg, -jnp.inf)

        vp = jnp.full((ROWS, 1), jnp.inf, jnp.float32)
        sp = jnp.full((ROWS, 1), -1, jnp.int32)
        newv, newi = [], []
        for _ in range(K):
            live_r = (vals < vp) | ((vals == vp) & (idxs > sp))
            mr = jnp.max(jnp.where(live_r, vals, -jnp.inf), axis=1, keepdims=True)
            live_c = (negm < vp) | ((negm == vp) & (col_ids > sp))
            mc = jnp.max(jnp.where(live_c, negm, -jnp.inf), axis=1, keepdims=True)
            m = jnp.maximum(mr, mc)
            selr = jnp.min(jnp.where((vals == m) & live_r, idxs, BIG),
                           axis=1, keepdims=True)
            selc = jnp.min(jnp.where((negm == m) & live_c,
                                     jnp.broadcast_to(col_ids, (ROWS, CHUNK)),
                                     BIG), axis=1, keepdims=True)
            sel = jnp.minimum(selr, selc)
            vp, sp = m, sel
            newv.append(m)
            newi.append(sel)
        vals = jnp.concatenate(
            newv + [jnp.full((ROWS, 32 - K), -jnp.inf, jnp.float32)], axis=1)
        idxs = jnp.concatenate(
            newi + [jnp.full((ROWS, 32 - K), BIG, jnp.int32)], axis=1)
        return vals, idxs

    vals0 = jnp.full((ROWS, 32), -jnp.inf, jnp.float32)
    idxs0 = jnp.full((ROWS, 32), BIG, jnp.int32)
    _, idxs = lax.fori_loop(0, cb_hi - cb_lo, chunk_step, (vals0, idxs0))
    idx_ref[:, 0:32] = jnp.clip(idxs, 0, N - 1)


def _run_topk(featp, rowmeta, wsub, wb, bias):
    return pl.pallas_call(
        _topk_body,
        grid=(NBLK,),
        in_specs=[
            pl.BlockSpec((N, 128), lambda g: (0, 0)),
            pl.BlockSpec((ROWS, 128), lambda g: (g, 0)),
            pl.BlockSpec((128, 128), lambda g: (0, 0)),
            pl.BlockSpec((128, 128), lambda g: (0, 0)),
            pl.BlockSpec((1, 128), lambda g: (0, 0)),
        ],
        out_specs=[
            pl.BlockSpec((ROWS, 128), lambda g: (g, 0)),
            pl.BlockSpec((ROWS, 128), lambda g: (g, 0)),
            pl.BlockSpec((ROWS, 128), lambda g: (g, 0)),
        ],
        out_shape=[
            jax.ShapeDtypeStruct((N, 128), jnp.int32),
            jax.ShapeDtypeStruct((N, 128), jnp.float32),
            jax.ShapeDtypeStruct((N, 128), jnp.float32),
        ],
    )(featp, rowmeta, wsub, wb, bias)


def _gather_max_sc(p128, idxT, c, F):
    nf = F // 16

    def body(p_hbm, idx_hbm, c_hbm, out_hbm,
             ic0, ic1, g0, g1, acc, cbuf, obuf, sem0, sem1):
        wid = lax.axis_index("s") * SC_NC + lax.axis_index("c")

        def build_col(j, dst, base):
            pltpu.sync_copy(idx_hbm.at[j, pl.ds(base, GCH)], dst)

        def reduce_into_acc(gbuf):
            def red(r, _):
                for f in range(nf):
                    sl = pl.ds(f * 16, 16)
                    acc[r, sl] = jnp.maximum(acc[r, sl], gbuf[r, sl])
                return 0
            lax.fori_loop(0, GCH, red, 0)

        for ch in range(PER_W // GCH):
            base = wid * PER_W + ch * GCH
            pltpu.sync_copy(c_hbm.at[pl.ds(base, GCH)], cbuf)
            build_col(0, ic0, base)
            cp = pltpu.async_copy(p_hbm.at[ic0], acc, sem0)
            build_col(1, ic1, base)
            cpn = pltpu.async_copy(p_hbm.at[ic1], g1, sem1)
            cp.wait()
            bufs = (g0, g1)
            ics = (ic0, ic1)
            sems = (sem0, sem1)
            for j in range(2, K):
                par = j % 2
                build_col(j, ics[par], base)
                cp = pltpu.async_copy(p_hbm.at[ics[par]], bufs[par], sems[par])
                cpn.wait()
                reduce_into_acc(bufs[1 - par])
                cpn = cp
            cpn.wait()
            reduce_into_acc(bufs[(K - 1) % 2])

            def finish(r, _):
                for f in range(nf):
                    sl = pl.ds(f * 16, 16)
                    obuf[r, sl] = jnp.maximum(acc[r, sl] + cbuf[r, sl], 0.0)
                return 0

            lax.fori_loop(0, GCH, finish, 0)
            pltpu.sync_copy(obuf, out_hbm.at[pl.ds(base, GCH)])

    fn = pl.kernel(
        body,
        mesh=plsc.VectorSubcoreMesh(core_axis_name="c", subcore_axis_name="s"),
        out_type=jax.ShapeDtypeStruct((N, F), jnp.float32),
        scratch_types=[
            pltpu.VMEM((GCH,), jnp.int32),
            pltpu.VMEM((GCH,), jnp.int32),
            pltpu.VMEM((GCH, 128), jnp.float32),
            pltpu.VMEM((GCH, 128), jnp.float32),
            pltpu.VMEM((GCH, 128), jnp.float32),
            pltpu.VMEM((GCH, F), jnp.float32),
            pltpu.VMEM((GCH, F), jnp.float32),
            pltpu.SemaphoreType.DMA,
            pltpu.SemaphoreType.DMA,
        ],
    )
    return fn(p128, idxT, c)


def _head_body(h_ref, wl_ref, bl_ref, wm1_ref, bm1_ref, wm2_ref, bm2_ref,
               wc_ref, bc_ref, out_ref):
    h = h_ref[...]
    h = jnp.maximum(jnp.dot(h, wl_ref[...],
                            preferred_element_type=jnp.float32) + bl_ref[...], 0.0)
    h = jnp.maximum(jnp.dot(h, wm1_ref[...],
                            preferred_element_type=jnp.float32) + bm1_ref[...], 0.0)
    h = jnp.maximum(jnp.dot(h, wm2_ref[...],
                            preferred_element_type=jnp.float32) + bm2_ref[...], 0.0)
    logits = jnp.dot(h, wc_ref[...],
                     preferred_element_type=jnp.float32) + bc_ref[...]
    m = jnp.max(logits, axis=1, keepdims=True)
    s = logits - m
    lse = jnp.log(jnp.sum(jnp.exp(s), axis=1, keepdims=True))
    out_ref[...] = s - lse


def _run_head(hcat, wl, bl, wm1, bm1, wm2, bm2, wc, bc):
    rb = 512
    return pl.pallas_call(
        _head_body,
        grid=(N // rb,),
        in_specs=[
            pl.BlockSpec((rb, 144), lambda g: (g, 0)),
            pl.BlockSpec((144, 1024), lambda g: (0, 0)),
            pl.BlockSpec((1, 1024), lambda g: (0, 0)),
            pl.BlockSpec((1024, 256), lambda g: (0, 0)),
            pl.BlockSpec((1, 256), lambda g: (0, 0)),
            pl.BlockSpec((256, 128), lambda g: (0, 0)),
            pl.BlockSpec((1, 128), lambda g: (0, 0)),
            pl.BlockSpec((128, 128), lambda g: (0, 0)),
            pl.BlockSpec((1, 128), lambda g: (0, 0)),
        ],
        out_specs=pl.BlockSpec((rb, 128), lambda g: (g, 0)),
        out_shape=jax.ShapeDtypeStruct((N, 128), jnp.float32),
    )(hcat, wl, bl, wm1, bm1, wm2, bm2, wc, bc)


def _pad2(a, r, c):
    return jnp.pad(a, ((0, r - a.shape[0]), (0, c - a.shape[1])))


def _layer_weights(W, b, cin):
    Wa, Wb = W[:cin], W[cin:]
    wsub = _pad2(Wa - Wb, 128, 128)
    wb = _pad2(Wb, 128, 128)
    bias = _pad2(b[None, :], 1, 128)
    return wsub, wb, bias


def _edge_layer(featp, rowmeta, W, b, cin, F):
    wsub, wb, bias = _layer_weights(W, b, cin)
    idxp, cpad, ppad = _run_topk(featp, rowmeta, wsub, wb, bias)
    return _gather_max_sc(ppad, idxp[:, :K].T, cpad[:, :F], F)


def kernel(x, batch, W1, b1, W2, b2, W3, b3, Wl, bl, Wm1, bm1, Wm2, bm2, Wc, bc):
    batch = batch.astype(jnp.int32)
    ar = jnp.arange(4, dtype=jnp.int32)
    seg_lo = jnp.searchsorted(batch, ar, side="left").astype(jnp.int32)
    seg_hi = jnp.searchsorted(batch, ar, side="right").astype(jnp.int32)
    lo = seg_lo[batch]
    hi = seg_hi[batch]
    rowmeta = jnp.zeros((N, 128), jnp.int32)
    rowmeta = rowmeta.at[:, 0].set(lo).at[:, 1].set(hi)

    featp = _pad2(x, N, 128)
    x1 = _edge_layer(featp, rowmeta, W1, b1, 3, 16)
    x2 = _edge_layer(_pad2(x1, N, 128), rowmeta, W2, b2, 9, 32)
    x3 = _edge_layer(_pad2(x2, N, 128), rowmeta, W3, b3, 27, 96)

    hcat = jnp.concatenate([x1, x2, x3], axis=1)
    wl = jnp.zeros((144, 1024), jnp.float32)
    wl = wl.at[0:9].set(Wl[0:9]).at[16:43].set(Wl[9:36]).at[48:129].set(Wl[36:117])
    blp = bl[None, :]
    wm1 = Wm1
    bm1p = bm1[None, :]
    wm2 = Wm2
    bm2p = bm2[None, :]
    wc = _pad2(Wc, 128, 128)
    bcp = jnp.full((1, 128), -1e30, jnp.float32).at[0, :40].set(bc)

    out = _run_head(hcat, wl, blp, wm1, bm1p, wm2, bm2p, wc, bcp)
    return out[:, :40]


# speedup vs baseline: 1.8657x; 1.8657x over previous
import functools

import jax
import jax.numpy as jnp
from jax import lax
from jax.experimental import pallas as pl
from jax.experimental.pallas import tpu as pltpu
from jax.experimental.pallas import tpu_sc as plsc

N = 8192
K = 20
ROWS = 256
CHUNK = 512
NCHUNK = N // CHUNK
NBLK = N // ROWS
BIG = 2 ** 30

SC_NC = 2
SC_NS = 16
SC_NW = SC_NC * SC_NS
PER_W = N // SC_NW
GCH = 128


def _topk_body(featp_ref, rowmeta_ref, wsub_ref, wb_ref, b_ref,
               idx_ref, c_ref, p_ref):
    g = pl.program_id(0)
    feat = featp_ref[pl.ds(g * ROWS, ROWS), :]
    c_ref[...] = jnp.dot(feat, wsub_ref[...],
                         preferred_element_type=jnp.float32) + b_ref[...]
    p_ref[...] = jnp.dot(feat, wb_ref[...],
                         preferred_element_type=jnp.float32)

    lo = rowmeta_ref[:, 0:1]
    hi = rowmeta_ref[:, 1:2]
    row_ids = g * ROWS + lax.broadcasted_iota(jnp.int32, (ROWS, 1), 0)

    cb_lo = jnp.min(lo) // CHUNK
    cb_hi = (jnp.max(hi) + CHUNK - 1) // CHUNK
    ones = jnp.full((1, 128), 1.0, jnp.float32)

    def chunk_step(t, carry):
        vals, idxs = carry
        cb = cb_lo + t
        fchunk = featp_ref[pl.ds(cb * CHUNK, CHUNK), :]
        dot = lax.dot_general(feat, fchunk, (((1,), (1,)), ((), ())),
                              preferred_element_type=jnp.float32)
        sqj = lax.dot_general(ones, fchunk * fchunk, (((1,), (1,)), ((), ())),
                              preferred_element_type=jnp.float32)
        neg = 2.0 * dot - sqj
        col_ids = cb * CHUNK + lax.broadcasted_iota(jnp.int32, (1, CHUNK), 1)
        valid = (col_ids >= lo) & (col_ids < hi) & (col_ids != row_ids)
        negm = jnp.where(valid, neg, -jnp.inf)

        cat_v = jnp.concatenate([vals, negm], axis=1)
        cat_i = jnp.concatenate(
            [idxs, jnp.broadcast_to(col_ids, (ROWS, CHUNK))], axis=1)
        newv, newi = [], []
        for _ in range(K):
            m = jnp.max(cat_v, axis=1, keepdims=True)
            ism = cat_v == m
            sel = jnp.min(jnp.where(ism, cat_i, BIG), axis=1, keepdims=True)
            cat_v = jnp.where(ism, -jnp.inf, cat_v)
            newv.append(m)
            newi.append(sel)
        vals = jnp.concatenate(
            newv + [jnp.full((ROWS, 32 - K), -jnp.inf, jnp.float32)], axis=1)
        idxs = jnp.concatenate(
            newi + [jnp.full((ROWS, 32 - K), BIG, jnp.int32)], axis=1)
        return vals, idxs

    vals0 = jnp.full((ROWS, 32), -jnp.inf, jnp.float32)
    idxs0 = jnp.full((ROWS, 32), BIG, jnp.int32)
    _, idxs = lax.fori_loop(0, cb_hi - cb_lo, chunk_step, (vals0, idxs0))
    idx_ref[:, 0:32] = jnp.clip(idxs, 0, N - 1)


def _run_topk(featp, rowmeta, wsub, wb, bias):
    return pl.pallas_call(
        _topk_body,
        grid=(NBLK,),
        in_specs=[
            pl.BlockSpec((N, 128), lambda g: (0, 0)),
            pl.BlockSpec((ROWS, 128), lambda g: (g, 0)),
            pl.BlockSpec((128, 128), lambda g: (0, 0)),
            pl.BlockSpec((128, 128), lambda g: (0, 0)),
            pl.BlockSpec((1, 128), lambda g: (0, 0)),
        ],
        out_specs=[
            pl.BlockSpec((ROWS, 128), lambda g: (g, 0)),
            pl.BlockSpec((ROWS, 128), lambda g: (g, 0)),
            pl.BlockSpec((ROWS, 128), lambda g: (g, 0)),
        ],
        out_shape=[
            jax.ShapeDtypeStruct((N, 128), jnp.int32),
            jax.ShapeDtypeStruct((N, 128), jnp.float32),
            jax.ShapeDtypeStruct((N, 128), jnp.float32),
        ],
    )(featp, rowmeta, wsub, wb, bias)


def _gather_max_sc(p128, idxT, c, F):
    nf = F // 16

    def body(p_hbm, idx_hbm, c_hbm, out_hbm,
             ic0, ic1, g0, g1, acc, cbuf, obuf, sem0, sem1):
        wid = lax.axis_index("s") * SC_NC + lax.axis_index("c")

        def build_col(j, dst, base):
            pltpu.sync_copy(idx_hbm.at[j, pl.ds(base, GCH)], dst)

        def reduce_into_acc(gbuf):
            def red(r, _):
                for f in range(nf):
                    sl = pl.ds(f * 16, 16)
                    acc[r, sl] = jnp.maximum(acc[r, sl], gbuf[r, sl])
                return 0
            lax.fori_loop(0, GCH, red, 0)

        for ch in range(PER_W // GCH):
            base = wid * PER_W + ch * GCH
            pltpu.sync_copy(c_hbm.at[pl.ds(base, GCH)], cbuf)
            build_col(0, ic0, base)
            cp = pltpu.async_copy(p_hbm.at[ic0], acc, sem0)
            build_col(1, ic1, base)
            cpn = pltpu.async_copy(p_hbm.at[ic1], g1, sem1)
            cp.wait()
            bufs = (g0, g1)
            ics = (ic0, ic1)
            sems = (sem0, sem1)
            for j in range(2, K):
                par = j % 2
                build_col(j, ics[par], base)
                cp = pltpu.async_copy(p_hbm.at[ics[par]], bufs[par], sems[par])
                cpn.wait()
                reduce_into_acc(bufs[1 - par])
                cpn = cp
            cpn.wait()
            reduce_into_acc(bufs[(K - 1) % 2])

            def finish(r, _):
                for f in range(nf):
                    sl = pl.ds(f * 16, 16)
                    obuf[r, sl] = jnp.maximum(acc[r, sl] + cbuf[r, sl], 0.0)
                return 0

            lax.fori_loop(0, GCH, finish, 0)
            pltpu.sync_copy(obuf, out_hbm.at[pl.ds(base, GCH)])

    fn = pl.kernel(
        body,
        mesh=plsc.VectorSubcoreMesh(core_axis_name="c", subcore_axis_name="s"),
        out_type=jax.ShapeDtypeStruct((N, F), jnp.float32),
        scratch_types=[
            pltpu.VMEM((GCH,), jnp.int32),
            pltpu.VMEM((GCH,), jnp.int32),
            pltpu.VMEM((GCH, 128), jnp.float32),
            pltpu.VMEM((GCH, 128), jnp.float32),
            pltpu.VMEM((GCH, 128), jnp.float32),
            pltpu.VMEM((GCH, F), jnp.float32),
            pltpu.VMEM((GCH, F), jnp.float32),
            pltpu.SemaphoreType.DMA,
            pltpu.SemaphoreType.DMA,
        ],
    )
    return fn(p128, idxT, c)


def _head_body(h_ref, wl_ref, bl_ref, wm1_ref, bm1_ref, wm2_ref, bm2_ref,
               wc_ref, bc_ref, out_ref):
    h = h_ref[...]
    h = jnp.maximum(jnp.dot(h, wl_ref[...],
                            preferred_element_type=jnp.float32) + bl_ref[...], 0.0)
    h = jnp.maximum(jnp.dot(h, wm1_ref[...],
                            preferred_element_type=jnp.float32) + bm1_ref[...], 0.0)
    h = jnp.maximum(jnp.dot(h, wm2_ref[...],
                            preferred_element_type=jnp.float32) + bm2_ref[...], 0.0)
    logits = jnp.dot(h, wc_ref[...],
                     preferred_element_type=jnp.float32) + bc_ref[...]
    m = jnp.max(logits, axis=1, keepdims=True)
    s = logits - m
    lse = jnp.log(jnp.sum(jnp.exp(s), axis=1, keepdims=True))
    out_ref[...] = s - lse


def _run_head(hcat, wl, bl, wm1, bm1, wm2, bm2, wc, bc):
    rb = 512
    return pl.pallas_call(
        _head_body,
        grid=(N // rb,),
        in_specs=[
            pl.BlockSpec((rb, 144), lambda g: (g, 0)),
            pl.BlockSpec((144, 1024), lambda g: (0, 0)),
            pl.BlockSpec((1, 1024), lambda g: (0, 0)),
            pl.BlockSpec((1024, 256), lambda g: (0, 0)),
            pl.BlockSpec((1, 256), lambda g: (0, 0)),
            pl.BlockSpec((256, 128), lambda g: (0, 0)),
            pl.BlockSpec((1, 128), lambda g: (0, 0)),
            pl.BlockSpec((128, 128), lambda g: (0, 0)),
            pl.BlockSpec((1, 128), lambda g: (0, 0)),
        ],
        out_specs=pl.BlockSpec((rb, 128), lambda g: (g, 0)),
        out_shape=jax.ShapeDtypeStruct((N, 128), jnp.float32),
    )(hcat, wl, bl, wm1, bm1, wm2, bm2, wc, bc)


def _pad2(a, r, c):
    return jnp.pad(a, ((0, r - a.shape[0]), (0, c - a.shape[1])))


def _layer_weights(W, b, cin):
    Wa, Wb = W[:cin], W[cin:]
    wsub = _pad2(Wa - Wb, 128, 128)
    wb = _pad2(Wb, 128, 128)
    bias = _pad2(b[None, :], 1, 128)
    return wsub, wb, bias


def _edge_layer(featp, rowmeta, W, b, cin, F):
    wsub, wb, bias = _layer_weights(W, b, cin)
    idxp, cpad, ppad = _run_topk(featp, rowmeta, wsub, wb, bias)
    return _gather_max_sc(ppad, idxp[:, :K].T, cpad[:, :F], F)


def kernel(x, batch, W1, b1, W2, b2, W3, b3, Wl, bl, Wm1, bm1, Wm2, bm2, Wc, bc):
    batch = batch.astype(jnp.int32)
    ar = jnp.arange(4, dtype=jnp.int32)
    seg_lo = jnp.searchsorted(batch, ar, side="left").astype(jnp.int32)
    seg_hi = jnp.searchsorted(batch, ar, side="right").astype(jnp.int32)
    lo = seg_lo[batch]
    hi = seg_hi[batch]
    rowmeta = jnp.zeros((N, 128), jnp.int32)
    rowmeta = rowmeta.at[:, 0].set(lo).at[:, 1].set(hi)

    featp = _pad2(x, N, 128)
    x1 = _edge_layer(featp, rowmeta, W1, b1, 3, 16)
    x2 = _edge_layer(_pad2(x1, N, 128), rowmeta, W2, b2, 9, 32)
    x3 = _edge_layer(_pad2(x2, N, 128), rowmeta, W3, b3, 27, 96)

    hcat = jnp.concatenate([x1, x2, x3], axis=1)
    wl = jnp.zeros((144, 1024), jnp.float32)
    wl = wl.at[0:9].set(Wl[0:9]).at[16:43].set(Wl[9:36]).at[48:129].set(Wl[36:117])
    blp = bl[None, :]
    wm1 = Wm1
    bm1p = bm1[None, :]
    wm2 = Wm2
    bm2p = bm2[None, :]
    wc = _pad2(Wc, 128, 128)
    bcp = jnp.full((1, 128), -1e30, jnp.float32).at[0, :40].set(bc)

    out = _run_head(hcat, wl, blp, wm1, bm1p, wm2, bm2p, wc, bcp)
    return out[:, :40]


# speedup vs baseline: 2.1773x; 1.1670x over previous
import functools

import jax
import jax.numpy as jnp
from jax import lax
from jax.experimental import pallas as pl
from jax.experimental.pallas import tpu as pltpu
from jax.experimental.pallas import tpu_sc as plsc

N = 8192
K = 20
ROWS = 256
CHUNK = 1024
NCHUNK = N // CHUNK
NBLK = N // ROWS
BIG = 2 ** 30

SC_NC = 2
SC_NS = 16
SC_NW = SC_NC * SC_NS
PER_W = N // SC_NW
GCH = 128


def _topk_body(featp_ref, rowmeta_ref, wsub_ref, wb_ref, b_ref,
               idx_ref, c_ref, p_ref):
    g = pl.program_id(0)
    feat = featp_ref[pl.ds(g * ROWS, ROWS), :]
    c_ref[...] = jnp.dot(feat, wsub_ref[...],
                         preferred_element_type=jnp.float32) + b_ref[...]
    p_ref[...] = jnp.dot(feat, wb_ref[...],
                         preferred_element_type=jnp.float32)

    lo = rowmeta_ref[:, 0:1]
    hi = rowmeta_ref[:, 1:2]
    row_ids = g * ROWS + lax.broadcasted_iota(jnp.int32, (ROWS, 1), 0)

    cb_lo = jnp.min(lo) // CHUNK
    cb_hi = (jnp.max(hi) + CHUNK - 1) // CHUNK
    ones = jnp.full((1, 128), 1.0, jnp.float32)

    def chunk_step(t, carry):
        vals, idxs = carry
        cb = cb_lo + t
        fchunk = featp_ref[pl.ds(cb * CHUNK, CHUNK), :]
        dot = lax.dot_general(feat, fchunk, (((1,), (1,)), ((), ())),
                              preferred_element_type=jnp.float32)
        sqj = lax.dot_general(ones, fchunk * fchunk, (((1,), (1,)), ((), ())),
                              preferred_element_type=jnp.float32)
        neg = 2.0 * dot - sqj
        col_ids = cb * CHUNK + lax.broadcasted_iota(jnp.int32, (1, CHUNK), 1)
        valid = (col_ids >= lo) & (col_ids < hi) & (col_ids != row_ids)
        negm = jnp.where(valid, neg, -jnp.inf)

        cat_v = jnp.concatenate([vals, negm], axis=1)
        cat_i = jnp.concatenate(
            [idxs, jnp.broadcast_to(col_ids, (ROWS, CHUNK))], axis=1)
        newv, newi = [], []
        for _ in range(K):
            m = jnp.max(cat_v, axis=1, keepdims=True)
            ism = cat_v == m
            sel = jnp.min(jnp.where(ism, cat_i, BIG), axis=1, keepdims=True)
            cat_v = jnp.where(ism, -jnp.inf, cat_v)
            newv.append(m)
            newi.append(sel)
        vals = jnp.concatenate(
            newv + [jnp.full((ROWS, 32 - K), -jnp.inf, jnp.float32)], axis=1)
        idxs = jnp.concatenate(
            newi + [jnp.full((ROWS, 32 - K), BIG, jnp.int32)], axis=1)
        return vals, idxs

    vals0 = jnp.full((ROWS, 32), -jnp.inf, jnp.float32)
    idxs0 = jnp.full((ROWS, 32), BIG, jnp.int32)
    _, idxs = lax.fori_loop(0, cb_hi - cb_lo, chunk_step, (vals0, idxs0))
    idx_ref[:, 0:32] = jnp.clip(idxs, 0, N - 1)


def _run_topk(featp, rowmeta, wsub, wb, bias):
    return pl.pallas_call(
        _topk_body,
        grid=(NBLK,),
        in_specs=[
            pl.BlockSpec((N, 128), lambda g: (0, 0)),
            pl.BlockSpec((ROWS, 128), lambda g: (g, 0)),
            pl.BlockSpec((128, 128), lambda g: (0, 0)),
            pl.BlockSpec((128, 128), lambda g: (0, 0)),
            pl.BlockSpec((1, 128), lambda g: (0, 0)),
        ],
        out_specs=[
            pl.BlockSpec((ROWS, 128), lambda g: (g, 0)),
            pl.BlockSpec((ROWS, 128), lambda g: (g, 0)),
            pl.BlockSpec((ROWS, 128), lambda g: (g, 0)),
        ],
        out_shape=[
            jax.ShapeDtypeStruct((N, 128), jnp.int32),
            jax.ShapeDtypeStruct((N, 128), jnp.float32),
            jax.ShapeDtypeStruct((N, 128), jnp.float32),
        ],
    )(featp, rowmeta, wsub, wb, bias)


def _gather_max_sc(p128, idxT, c, F):
    nf = F // 16

    def body(p_hbm, idx_hbm, c_hbm, out_hbm,
             ic0, ic1, g0, g1, acc, cbuf, obuf, sem0, sem1):
        wid = lax.axis_index("s") * SC_NC + lax.axis_index("c")

        def build_col(j, dst, base):
            pltpu.sync_copy(idx_hbm.at[j, pl.ds(base, GCH)], dst)

        def reduce_into_acc(gbuf):
            def red(r, _):
                for f in range(nf):
                    sl = pl.ds(f * 16, 16)
                    acc[r, sl] = jnp.maximum(acc[r, sl], gbuf[r, sl])
                return 0
            lax.fori_loop(0, GCH, red, 0)

        for ch in range(PER_W // GCH):
            base = wid * PER_W + ch * GCH
            pltpu.sync_copy(c_hbm.at[pl.ds(base, GCH)], cbuf)
            build_col(0, ic0, base)
            cp = pltpu.async_copy(p_hbm.at[ic0], acc, sem0)
            build_col(1, ic1, base)
            cpn = pltpu.async_copy(p_hbm.at[ic1], g1, sem1)
            cp.wait()
            bufs = (g0, g1)
            ics = (ic0, ic1)
            sems = (sem0, sem1)
            for j in range(2, K):
                par = j % 2
                build_col(j, ics[par], base)
                cp = pltpu.async_copy(p_hbm.at[ics[par]], bufs[par], sems[par])
                cpn.wait()
                reduce_into_acc(bufs[1 - par])
                cpn = cp
            cpn.wait()
            reduce_into_acc(bufs[(K - 1) % 2])

            def finish(r, _):
                for f in range(nf):
                    sl = pl.ds(f * 16, 16)
                    obuf[r, sl] = jnp.maximum(acc[r, sl] + cbuf[r, sl], 0.0)
                return 0

            lax.fori_loop(0, GCH, finish, 0)
            pltpu.sync_copy(obuf, out_hbm.at[pl.ds(base, GCH)])

    fn = pl.kernel(
        body,
        mesh=plsc.VectorSubcoreMesh(core_axis_name="c", subcore_axis_name="s"),
        out_type=jax.ShapeDtypeStruct((N, F), jnp.float32),
        scratch_types=[
            pltpu.VMEM((GCH,), jnp.int32),
            pltpu.VMEM((GCH,), jnp.int32),
            pltpu.VMEM((GCH, 128), jnp.float32),
            pltpu.VMEM((GCH, 128), jnp.float32),
            pltpu.VMEM((GCH, 128), jnp.float32),
            pltpu.VMEM((GCH, F), jnp.float32),
            pltpu.VMEM((GCH, F), jnp.float32),
            pltpu.SemaphoreType.DMA,
            pltpu.SemaphoreType.DMA,
        ],
    )
    return fn(p128, idxT, c)


def _head_body(h_ref, wl_ref, bl_ref, wm1_ref, bm1_ref, wm2_ref, bm2_ref,
               wc_ref, bc_ref, out_ref):
    h = h_ref[...]
    h = jnp.maximum(jnp.dot(h, wl_ref[...],
                            preferred_element_type=jnp.float32) + bl_ref[...], 0.0)
    h = jnp.maximum(jnp.dot(h, wm1_ref[...],
                            preferred_element_type=jnp.float32) + bm1_ref[...], 0.0)
    h = jnp.maximum(jnp.dot(h, wm2_ref[...],
                            preferred_element_type=jnp.float32) + bm2_ref[...], 0.0)
    logits = jnp.dot(h, wc_ref[...],
                     preferred_element_type=jnp.float32) + bc_ref[...]
    m = jnp.max(logits, axis=1, keepdims=True)
    s = logits - m
    lse = jnp.log(jnp.sum(jnp.exp(s), axis=1, keepdims=True))
    out_ref[...] = s - lse


def _run_head(hcat, wl, bl, wm1, bm1, wm2, bm2, wc, bc):
    rb = 512
    return pl.pallas_call(
        _head_body,
        grid=(N // rb,),
        in_specs=[
            pl.BlockSpec((rb, 144), lambda g: (g, 0)),
            pl.BlockSpec((144, 1024), lambda g: (0, 0)),
            pl.BlockSpec((1, 1024), lambda g: (0, 0)),
            pl.BlockSpec((1024, 256), lambda g: (0, 0)),
            pl.BlockSpec((1, 256), lambda g: (0, 0)),
            pl.BlockSpec((256, 128), lambda g: (0, 0)),
            pl.BlockSpec((1, 128), lambda g: (0, 0)),
            pl.BlockSpec((128, 128), lambda g: (0, 0)),
            pl.BlockSpec((1, 128), lambda g: (0, 0)),
        ],
        out_specs=pl.BlockSpec((rb, 128), lambda g: (g, 0)),
        out_shape=jax.ShapeDtypeStruct((N, 128), jnp.float32),
    )(hcat, wl, bl, wm1, bm1, wm2, bm2, wc, bc)


def _pad2(a, r, c):
    return jnp.pad(a, ((0, r - a.shape[0]), (0, c - a.shape[1])))


def _layer_weights(W, b, cin):
    Wa, Wb = W[:cin], W[cin:]
    wsub = _pad2(Wa - Wb, 128, 128)
    wb = _pad2(Wb, 128, 128)
    bias = _pad2(b[None, :], 1, 128)
    return wsub, wb, bias


def _edge_layer(featp, rowmeta, W, b, cin, F):
    wsub, wb, bias = _layer_weights(W, b, cin)
    idxp, cpad, ppad = _run_topk(featp, rowmeta, wsub, wb, bias)
    return _gather_max_sc(ppad, idxp[:, :K].T, cpad[:, :F], F)


def kernel(x, batch, W1, b1, W2, b2, W3, b3, Wl, bl, Wm1, bm1, Wm2, bm2, Wc, bc):
    batch = batch.astype(jnp.int32)
    ar = jnp.arange(4, dtype=jnp.int32)
    seg_lo = jnp.searchsorted(batch, ar, side="left").astype(jnp.int32)
    seg_hi = jnp.searchsorted(batch, ar, side="right").astype(jnp.int32)
    lo = seg_lo[batch]
    hi = seg_hi[batch]
    rowmeta = jnp.zeros((N, 128), jnp.int32)
    rowmeta = rowmeta.at[:, 0].set(lo).at[:, 1].set(hi)

    featp = _pad2(x, N, 128)
    x1 = _edge_layer(featp, rowmeta, W1, b1, 3, 16)
    x2 = _edge_layer(_pad2(x1, N, 128), rowmeta, W2, b2, 9, 32)
    x3 = _edge_layer(_pad2(x2, N, 128), rowmeta, W3, b3, 27, 96)

    hcat = jnp.concatenate([x1, x2, x3], axis=1)
    wl = jnp.zeros((144, 1024), jnp.float32)
    wl = wl.at[0:9].set(Wl[0:9]).at[16:43].set(Wl[9:36]).at[48:129].set(Wl[36:117])
    blp = bl[None, :]
    wm1 = Wm1
    bm1p = bm1[None, :]
    wm2 = Wm2
    bm2p = bm2[None, :]
    wc = _pad2(Wc, 128, 128)
    bcp = jnp.full((1, 128), -1e30, jnp.float32).at[0, :40].set(bc)

    out = _run_head(hcat, wl, blp, wm1, bm1p, wm2, bm2p, wc, bcp)
    return out[:, :40]
